# baseline (speedup 1.0000x reference)
import functools

import numpy as np
import jax
import jax.numpy as jnp
from jax import lax
from jax.experimental import pallas as pl
from jax.experimental.pallas import tpu as pltpu
from jax.experimental.pallas import tpu_sc as plsc

L = 16
T = 524288
F = 2
N_POINTS = 262144
BASE_RES = 16
SCALE = 1.3819128800392342
RES = [int(np.floor(BASE_RES * SCALE ** l)) for l in range(L)]
P1 = np.int32(2654435761 - 2 ** 32)
P2 = np.int32(805459861)
MASK = np.int32(T - 1)

NC, NS = 2, 16
NW = NC * NS
PPW = N_POINTS // NW
CHUNK = 1024
VG = CHUNK // 16
CI = VG * 128
NCH = PPW // CHUNK
G = N_POINTS // CHUNK


def _sc_body(pts_hbm, tab_hbm, out_hbm, pts_v, xyz_v, idx_v, r0_v, r1_v, enc_v, sem):
    c = lax.axis_index("c")
    s = lax.axis_index("s")
    wid = s * NC + c
    iota = lax.iota(jnp.int32, 16)
    iota3 = iota * 3

    def chunk_body(ch, carry):
        g = wid * NCH + ch
        pltpu.sync_copy(
            pts_hbm.at[pl.ds(pl.multiple_of(g * (3 * CHUNK), 3 * CHUNK), 3 * CHUNK)],
            pts_v)

        def xpose(v, cr):
            r = pl.multiple_of(v * 16, 16)
            b3 = v * 48
            xyz_v[0, pl.ds(r, 16)] = plsc.load_gather(pts_v, [iota3 + b3])
            xyz_v[1, pl.ds(r, 16)] = plsc.load_gather(pts_v, [iota3 + (b3 + 1)])
            xyz_v[2, pl.ds(r, 16)] = plsc.load_gather(pts_v, [iota3 + (b3 + 2)])
            return cr
        lax.fori_loop(0, VG, xpose, 0)

        for l in range(L):
            res = np.float32(RES[l])
            loff2 = np.int32(2 * l * T)

            def idx_body(v, cr, res=res, loff2=loff2):
                r = pl.multiple_of(v * 16, 16)
                b = pl.multiple_of(v * 128, 128)
                xs = xyz_v[0, pl.ds(r, 16)] * res
                ys = xyz_v[1, pl.ds(r, 16)] * res
                zs = xyz_v[2, pl.ds(r, 16)] * res
                xi = xs.astype(jnp.int32)
                yi = ys.astype(jnp.int32)
                zi = zs.astype(jnp.int32)
                hx0 = xi
                hx1 = xi + 1
                hy0 = yi * P1
                hy1 = hy0 + P1
                hz0 = zi * P2
                hz1 = hz0 + P2
                k = 0
                for hx in (hx0, hx1):
                    for hy in (hy0, hy1):
                        for hz in (hz0, hz1):
                            e = ((((hx ^ hy) ^ hz) & MASK) << 1) + loff2
                            idx_v[pl.ds(b + k * 16, 16)] = e
                            idx_v[pl.ds(CI + b + k * 16, 16)] = e + 1
                            k += 1
                return cr
            lax.fori_loop(0, VG, idx_body, 0)

            def fire(j, cr):
                b = pl.multiple_of(j * 128, 128)
                pltpu.make_async_copy(
                    tab_hbm.at[idx_v.at[pl.ds(b, 128)]],
                    r0_v.at[pl.ds(b, 128)], sem).start()
                pltpu.make_async_copy(
                    tab_hbm.at[idx_v.at[pl.ds(CI + b, 128)]],
                    r1_v.at[pl.ds(b, 128)], sem).start()
                return cr
            lax.fori_loop(0, VG, fire, 0)

            def drain(j, cr):
                b = pl.multiple_of(j * 128, 128)
                pltpu.make_async_copy(
                    tab_hbm.at[idx_v.at[pl.ds(b, 128)]],
                    r0_v.at[pl.ds(b, 128)], sem).wait()
                pltpu.make_async_copy(
                    tab_hbm.at[idx_v.at[pl.ds(CI + b, 128)]],
                    r1_v.at[pl.ds(b, 128)], sem).wait()
                return cr
            lax.fori_loop(0, VG, drain, 0)

            def interp(v, cr, res=res, l=l):
                r = pl.multiple_of(v * 16, 16)
                b = pl.multiple_of(v * 128, 128)
                xs = xyz_v[0, pl.ds(r, 16)] * res
                ys = xyz_v[1, pl.ds(r, 16)] * res
                zs = xyz_v[2, pl.ds(r, 16)] * res
                wx1 = xs - xs.astype(jnp.int32).astype(jnp.float32)
                wy1 = ys - ys.astype(jnp.int32).astype(jnp.float32)
                wz1 = zs - zs.astype(jnp.int32).astype(jnp.float32)
                wx0 = 1.0 - wx1
                wy0 = 1.0 - wy1
                wz0 = 1.0 - wz1
                wxy = (wx0 * wy0, wx0 * wy1, wx1 * wy0, wx1 * wy1)
                wz = (wz0, wz1)
                acc0 = jnp.zeros((16,), jnp.float32)
                acc1 = jnp.zeros((16,), jnp.float32)
                for k in range(8):
                    wk = wxy[k >> 1] * wz[k & 1]
                    f0 = r0_v[pl.ds(b + k * 16, 16)]
                    f1 = r1_v[pl.ds(b + k * 16, 16)]
                    acc0 = acc0 + wk * f0
                    acc1 = acc1 + wk * f1
                enc_v[2 * l, pl.ds(r, 16)] = acc0
                enc_v[2 * l + 1, pl.ds(r, 16)] = acc1
                return cr
            lax.fori_loop(0, VG, interp, 0)

        pltpu.sync_copy(enc_v, out_hbm.at[g])
        return carry

    lax.fori_loop(0, NCH, chunk_body, 0)


@functools.partial(
    pl.kernel,
    out_type=jax.ShapeDtypeStruct((G, 2 * L, CHUNK), jnp.float32),
    mesh=plsc.VectorSubcoreMesh(core_axis_name="c", subcore_axis_name="s"),
    compiler_params=pltpu.CompilerParams(needs_layout_passes=False),
    scratch_types=[
        pltpu.VMEM((3 * CHUNK,), jnp.float32),
        pltpu.VMEM((3, CHUNK), jnp.float32),
        pltpu.VMEM((2 * CI,), jnp.int32),
        pltpu.VMEM((CI,), jnp.float32),
        pltpu.VMEM((CI,), jnp.float32),
        pltpu.VMEM((2 * L, CHUNK), jnp.float32),
        pltpu.SemaphoreType.DMA,
    ],
)
def _sc_encode(pts_hbm, tab_hbm, out_hbm, pts_v, xyz_v, idx_v, r0_v, r1_v, enc_v, sem):
    _sc_body(pts_hbm, tab_hbm, out_hbm, pts_v, xyz_v, idx_v, r0_v, r1_v, enc_v, sem)


def _mlp_body(e_ref, w0_ref, w1_ref, w2_ref, o_ref):
    dn = (((0,), (0,)), ((), ()))
    h = lax.dot_general(w0_ref[...], e_ref[...], dn,
                        preferred_element_type=jnp.float32)
    h = jnp.maximum(h, 0.0)
    h = lax.dot_general(w1_ref[...], h, dn, preferred_element_type=jnp.float32)
    h = jnp.maximum(h, 0.0)
    o_ref[...] = lax.dot_general(w2_ref[...], h, dn,
                                 preferred_element_type=jnp.float32)


def kernel(inputs, table, W0, W1, W2):
    tabf = table.reshape(L * T * F)
    ptsf = inputs.reshape(N_POINTS * 3)
    enc = _sc_encode(ptsf, tabf)
    enc2 = enc.reshape(G * 2 * L, CHUNK)
    w2p = jnp.pad(W2, ((0, 0), (0, 7)))
    out8 = pl.pallas_call(
        _mlp_body,
        grid=(G,),
        in_specs=[
            pl.BlockSpec((2 * L, CHUNK), lambda i: (i, 0)),
            pl.BlockSpec((2 * L, 64), lambda i: (0, 0)),
            pl.BlockSpec((64, 64), lambda i: (0, 0)),
            pl.BlockSpec((64, 8), lambda i: (0, 0)),
        ],
        out_specs=pl.BlockSpec((8, CHUNK), lambda i: (i, 0)),
        out_shape=jax.ShapeDtypeStruct((G * 8, CHUNK), jnp.float32),
    )(enc2, W0, W1, w2p)
    return out8.reshape(G, 8, CHUNK)[:, 0, :].reshape(N_POINTS, 1)
